# Initial kernel scaffold; baseline (speedup 1.0000x reference)
import jax
import jax.numpy as jnp
from jax.experimental import pallas as pl


def kernel(image_features, graph_features, batch):
    raise NotImplementedError("write your pallas kernel here")


# speedup vs baseline: 1.1543x; 1.1543x over previous
import functools

import jax
import jax.numpy as jnp
import numpy as np
from jax import lax
from jax.experimental import pallas as pl
from jax.experimental.pallas import tpu as pltpu
from jax.experimental.pallas import tpu_sc as plsc

_C = 64
_H = 64
_HWD = _H * _H * _H
_N = 200000
_NC, _NS = 2, 16
_NW = _NC * _NS
_L = 16
_P = 64
_CHUNKS = 98
_PPT = _P * _CHUNKS
_NPAD = _PPT * _NW
_OFFS = (0, 1, 64, 65, 4096, 4097, 4160, 4161)

_PERM_BLOCK = np.stack([np.arange(16), 16 + np.arange(16)], 1).reshape(-1)
_PERM = np.concatenate([_PERM_BLOCK, 32 + _PERM_BLOCK])

_MESH = plsc.VectorSubcoreMesh(
    core_axis_name="c", subcore_axis_name="s",
    num_cores=_NC, num_subcores=_NS)


def _issue(c, b, xv, yv, zv, idx_v, w_v, rows_v, table, sem):
  one = jnp.full((_L,), 1.0, jnp.float32)
  zero = jnp.zeros((_L,), jnp.float32)
  for g in range(_P // _L):
    s = c * _P + g * _L
    x = xv[pl.ds(s, _L)]
    y = yv[pl.ds(s, _L)]
    z = zv[pl.ds(s, _L)]
    xi = x.astype(jnp.int32)
    yi = y.astype(jnp.int32)
    zi = z.astype(jnp.int32)
    xf = xi.astype(jnp.float32)
    yf = yi.astype(jnp.float32)
    zf = zi.astype(jnp.float32)
    xhi = x - xf
    yhi = y - yf
    zhi = z - zf
    xlo = jnp.where(x > xf, one - xhi, zero)
    ylo = jnp.where(y > yf, one - yhi, zero)
    zlo = jnp.where(z > zf, one - zhi, zero)
    cell = xi * 4096 + yi * 64 + zi
    for c8, off in enumerate(_OFFS):
      idx_v[b, c8, pl.ds(g * _L, _L)] = cell + off
    w_v[b, 0, pl.ds(g * _L, _L)] = xlo
    w_v[b, 1, pl.ds(g * _L, _L)] = xhi
    w_v[b, 2, pl.ds(g * _L, _L)] = ylo
    w_v[b, 3, pl.ds(g * _L, _L)] = yhi
    w_v[b, 4, pl.ds(g * _L, _L)] = zlo
    w_v[b, 5, pl.ds(g * _L, _L)] = zhi
  for c8 in range(8):
    pltpu.async_copy(table.at[idx_v.at[b, c8]], rows_v.at[b, c8], sem)


def _combine(c, b, base, idx_v, w_v, rows_v, out_v, out, table, sem):
  for c8 in range(8):
    pltpu.make_async_copy(
        table.at[idx_v.at[b, c8]], rows_v.at[b, c8], sem).wait()

  def group(g, carry):
    wch = [w_v[b, j, pl.ds(g * _L, _L)] for j in range(6)]
    for lane in range(_L):
      p = g * _L + lane
      bw = [jnp.broadcast_to(wch[j][lane], (_L,)) for j in range(6)]
      xlo, xhi, ylo, yhi, zlo, zhi = bw
      for k2 in range(2):
        q = [plsc.unpack(rows_v[b, c8, p, pl.ds(k2 * 2 * _L, 2 * _L)],
                         format=plsc.PackFormat.INTERLEAVED)
             for c8 in range(8)]
        for h in range(2):
          t00 = q[0][h] * zlo + q[1][h] * zhi
          t01 = q[2][h] * zlo + q[3][h] * zhi
          t10 = q[4][h] * zlo + q[5][h] * zhi
          t11 = q[6][h] * zlo + q[7][h] * zhi
          u0 = t00 * ylo + t01 * yhi
          u1 = t10 * ylo + t11 * yhi
          out_v[p, pl.ds((k2 * 2 + h) * _L, _L)] = u0 * xlo + u1 * xhi
    return carry

  lax.fori_loop(0, _P // _L, group, 0)
  row = base + c * _P

  @pl.when(row < _N)
  def _():
    pltpu.sync_copy(out_v, out.at[pl.ds(row, _P)])


@functools.partial(
    pl.kernel,
    out_type=jax.ShapeDtypeStruct((_N, _C), jnp.float32),
    mesh=_MESH,
    scratch_types=dict(
        xv=pltpu.VMEM((_PPT,), jnp.float32),
        yv=pltpu.VMEM((_PPT,), jnp.float32),
        zv=pltpu.VMEM((_PPT,), jnp.float32),
        idx_v=pltpu.VMEM((2, 8, _P), jnp.int32),
        w_v=pltpu.VMEM((2, 6, _P), jnp.float32),
        rows_v=pltpu.VMEM((2, 8, _P, _C), jnp.bfloat16),
        out_v=pltpu.VMEM((_P, _C), jnp.float32),
        sem0=pltpu.SemaphoreType.DMA,
        sem1=pltpu.SemaphoreType.DMA,
    ),
    compiler_params=pltpu.CompilerParams(
        use_tc_tiling_on_sc=False, needs_layout_passes=False),
)
def _sc_project(table, xs, ys, zs, out, xv, yv, zv, idx_v, w_v, rows_v,
                out_v, sem0, sem1):
  wid = lax.axis_index("s") * _NC + lax.axis_index("c")
  base = wid * _PPT
  pltpu.sync_copy(xs.at[pl.ds(base, _PPT)], xv)
  pltpu.sync_copy(ys.at[pl.ds(base, _PPT)], yv)
  pltpu.sync_copy(zs.at[pl.ds(base, _PPT)], zv)
  sems = (sem0, sem1)
  _issue(0, 0, xv, yv, zv, idx_v, w_v, rows_v, table, sems[0])

  def pair(t, carry):
    cc = t * 2
    for b in range(2):
      c = cc + b

      @pl.when(c + 1 < _CHUNKS)
      def _():
        _issue(c + 1, 1 - b, xv, yv, zv, idx_v, w_v, rows_v, table,
               sems[1 - b])

      _combine(c, b, base, idx_v, w_v, rows_v, out_v, out, table, sems[b])
    return carry

  lax.fori_loop(0, _CHUNKS // 2, pair, 0)


def kernel(image_features, graph_features, batch):
  table = image_features[0].transpose(1, 2, 3, 0).reshape(_HWD, _C)
  table = table[:, _PERM].astype(jnp.bfloat16)
  gp = jnp.concatenate(
      [graph_features,
       jnp.zeros((_NPAD - _N, 3), graph_features.dtype)], axis=0)
  xs = gp[:, 0]
  ys = gp[:, 1]
  zs = gp[:, 2]
  return _sc_project(table, xs, ys, zs)


# speedup vs baseline: 1.9489x; 1.6884x over previous
import functools

import jax
import jax.numpy as jnp
from jax import lax
from jax.experimental import pallas as pl
from jax.experimental.pallas import tpu as pltpu
from jax.experimental.pallas import tpu_sc as plsc

_C = 64
_H = 64
_HWD = _H * _H * _H
_N = 200000
_NC, _NS = 2, 16
_NW = _NC * _NS
_L = 16
_P = 64
_CHUNKS = 98
_PPT = _P * _CHUNKS
_NPAD = _PPT * _NW
_OFFS = (0, 1, 64, 65, 4096, 4097, 4160, 4161)

_MESH = plsc.VectorSubcoreMesh(
    core_axis_name="c", subcore_axis_name="s",
    num_cores=_NC, num_subcores=_NS)


def _issue(c, b, xv, yv, zv, idx_v, w_v, rows_v, table, sem):
  one = jnp.full((_L,), 1.0, jnp.float32)
  zero = jnp.zeros((_L,), jnp.float32)
  for g in range(_P // _L):
    s = c * _P + g * _L
    x = xv[pl.ds(s, _L)]
    y = yv[pl.ds(s, _L)]
    z = zv[pl.ds(s, _L)]
    xi = x.astype(jnp.int32)
    yi = y.astype(jnp.int32)
    zi = z.astype(jnp.int32)
    xf = xi.astype(jnp.float32)
    yf = yi.astype(jnp.float32)
    zf = zi.astype(jnp.float32)
    xhi = x - xf
    yhi = y - yf
    zhi = z - zf
    xlo = jnp.where(x > xf, one - xhi, zero)
    ylo = jnp.where(y > yf, one - yhi, zero)
    zlo = jnp.where(z > zf, one - zhi, zero)
    cell = xi * 4096 + yi * 64 + zi
    for c8, off in enumerate(_OFFS):
      idx_v[b, c8, pl.ds(g * _L, _L)] = cell + off
    w_v[b, 0, pl.ds(g * _L, _L)] = xlo
    w_v[b, 1, pl.ds(g * _L, _L)] = xhi
    w_v[b, 2, pl.ds(g * _L, _L)] = ylo
    w_v[b, 3, pl.ds(g * _L, _L)] = yhi
    w_v[b, 4, pl.ds(g * _L, _L)] = zlo
    w_v[b, 5, pl.ds(g * _L, _L)] = zhi
  for c8 in range(8):
    pltpu.async_copy(table.at[idx_v.at[b, c8]], rows_v.at[b, c8], sem)


def _combine(c, b, base, idx_v, w_v, rows_v, out_v, out, table, sem):
  for c8 in range(8):
    pltpu.make_async_copy(
        table.at[idx_v.at[b, c8]], rows_v.at[b, c8], sem).wait()

  def group(g, carry):
    wch = [w_v[b, j, pl.ds(g * _L, _L)] for j in range(6)]
    for lane in range(_L):
      p = g * _L + lane
      bw = [jnp.broadcast_to(wch[j][lane], (_L,)) for j in range(6)]
      xlo, xhi, ylo, yhi, zlo, zhi = bw
      for k in range(_C // _L):
        q = [rows_v[b, c8, p, pl.ds(k * _L, _L)] for c8 in range(8)]
        t00 = q[0] * zlo + q[1] * zhi
        t01 = q[2] * zlo + q[3] * zhi
        t10 = q[4] * zlo + q[5] * zhi
        t11 = q[6] * zlo + q[7] * zhi
        u0 = t00 * ylo + t01 * yhi
        u1 = t10 * ylo + t11 * yhi
        out_v[pl.ds(p * _C + k * _L, _L)] = u0 * xlo + u1 * xhi
    return carry

  lax.fori_loop(0, _P // _L, group, 0)
  row = base + c * _P

  @pl.when(row < _N)
  def _():
    pltpu.sync_copy(out_v, out.at[pl.ds(row * _C, _P * _C)])


@functools.partial(
    pl.kernel,
    out_type=jax.ShapeDtypeStruct((_N * _C,), jnp.float32),
    mesh=_MESH,
    scratch_types=dict(
        xv=pltpu.VMEM((_PPT,), jnp.float32),
        yv=pltpu.VMEM((_PPT,), jnp.float32),
        zv=pltpu.VMEM((_PPT,), jnp.float32),
        idx_v=pltpu.VMEM((2, 8, _P), jnp.int32),
        w_v=pltpu.VMEM((2, 6, _P), jnp.float32),
        rows_v=pltpu.VMEM((2, 8, _P, _C), jnp.float32),
        out_v=pltpu.VMEM((_P * _C,), jnp.float32),
        sem0=pltpu.SemaphoreType.DMA,
        sem1=pltpu.SemaphoreType.DMA,
    ),
    compiler_params=pltpu.CompilerParams(use_tc_tiling_on_sc=False),
)
def _sc_project(table, xs, ys, zs, out, xv, yv, zv, idx_v, w_v, rows_v,
                out_v, sem0, sem1):
  wid = lax.axis_index("s") * _NC + lax.axis_index("c")
  base = wid * _PPT
  pltpu.sync_copy(xs.at[pl.ds(base, _PPT)], xv)
  pltpu.sync_copy(ys.at[pl.ds(base, _PPT)], yv)
  pltpu.sync_copy(zs.at[pl.ds(base, _PPT)], zv)
  sems = (sem0, sem1)
  _issue(0, 0, xv, yv, zv, idx_v, w_v, rows_v, table, sems[0])

  def pair(t, carry):
    cc = t * 2
    for b in range(2):
      c = cc + b

      @pl.when(c + 1 < _CHUNKS)
      def _():
        _issue(c + 1, 1 - b, xv, yv, zv, idx_v, w_v, rows_v, table,
               sems[1 - b])

      _combine(c, b, base, idx_v, w_v, rows_v, out_v, out, table, sems[b])
    return carry

  lax.fori_loop(0, _CHUNKS // 2, pair, 0)


def kernel(image_features, graph_features, batch):
  table = image_features[0].transpose(1, 2, 3, 0).reshape(_HWD, _C)
  gp = jnp.concatenate(
      [graph_features,
       jnp.zeros((_NPAD - _N, 3), graph_features.dtype)], axis=0)
  xs = gp[:, 0]
  ys = gp[:, 1]
  zs = gp[:, 2]
  out = _sc_project(table, xs, ys, zs)
  return out.reshape(_N, _C)
